# baseline (speedup 1.0000x reference)
import jax
import jax.numpy as jnp
from jax.experimental import pallas as pl
from jax.experimental.pallas import tpu as pltpu

_B = 2048
_K = 4


def _seg_kernel(lo_c, hi_c, n_actual, *refs):
    node_refs = refs[:_K]
    ids_refs = refs[_K:2 * _K]
    glob_ref, w1_ref, w2_ref, b_ref, out_ref, accum_ref = refs[2 * _K:]

    t = pl.program_id(0)
    nsteps = pl.num_programs(0)

    @pl.when(t == 0)
    def _init():
        accum_ref[...] = jnp.full_like(accum_ref[...], -jnp.inf)

    for k in range(_K):
        blk = k * nsteps + t
        row = blk * _B + jax.lax.broadcasted_iota(jnp.int32, (_B, 1), 0)
        row_ok = row < n_actual[0]
        ids_vec = ids_refs[k][...]
        nodes = node_refs[k][...]

        def body(g, _, ids_vec=ids_vec, row_ok=row_ok, nodes=nodes):
            mask = (ids_vec == g) & row_ok
            vals = jnp.where(mask, nodes, -jnp.inf)
            bmax = jnp.max(vals, axis=0, keepdims=True)
            cur = accum_ref[pl.ds(g, 1), :]
            accum_ref[pl.ds(g, 1), :] = jnp.maximum(cur, bmax)
            return 0

        jax.lax.fori_loop(lo_c[blk], hi_c[blk] + 1, body, 0)

    @pl.when(t == nsteps - 1)
    def _fin():
        gpad = accum_ref.shape[0]
        gidx = jax.lax.broadcasted_iota(jnp.int32, (gpad, 1), 0)
        nseg = glob_ref.shape[0]
        acc = jnp.where(gidx < nseg, accum_ref[...], 0.0)
        out = jnp.dot(acc, w1_ref[...], preferred_element_type=jnp.float32)
        out += jnp.dot(glob_ref[...], w2_ref[...],
                       preferred_element_type=jnp.float32)
        out_ref[...] = out + b_ref[...]


def kernel(nodes, segment_ids, globals_, W, b):
    n, d = nodes.shape
    g, dg = globals_.shape
    mlp = W.shape[1]
    gpad = 128
    nsteps = (n + _K * _B - 1) // (_K * _B)
    nblocks = _K * nsteps
    npad = nblocks * _B

    ids = segment_ids.astype(jnp.int32)
    ids_pad = jnp.full((npad,), g - 1, jnp.int32).at[:n].set(ids)
    ids_2d = ids_pad.reshape(npad, 1)
    lo_c = ids_pad[::_B]
    hi_c = ids_pad[_B - 1::_B]
    n_actual = jnp.full((1,), n, jnp.int32)

    glob_pad = jnp.zeros((gpad, dg), jnp.float32).at[:g].set(globals_)
    w1 = W[:d]
    w2 = W[d:]
    b2 = b.reshape(1, mlp)

    last_real = (n - 1) // _B

    def stripe_spec(k, width):
        return pl.BlockSpec(
            (_B, width),
            lambda t, lc, hc, na, k=k: (
                jnp.minimum(k * nsteps + t, last_real), 0))

    in_specs = (
        [stripe_spec(k, d) for k in range(_K)]
        + [stripe_spec(k, 1) for k in range(_K)]
        + [
            pl.BlockSpec((gpad, dg), lambda t, lc, hc, na: (0, 0)),
            pl.BlockSpec((d, mlp), lambda t, lc, hc, na: (0, 0)),
            pl.BlockSpec((dg, mlp), lambda t, lc, hc, na: (0, 0)),
            pl.BlockSpec((1, mlp), lambda t, lc, hc, na: (0, 0)),
        ])

    grid_spec = pltpu.PrefetchScalarGridSpec(
        num_scalar_prefetch=3,
        grid=(nsteps,),
        in_specs=in_specs,
        out_specs=pl.BlockSpec((gpad, mlp), lambda t, lc, hc, na: (0, 0)),
        scratch_shapes=[pltpu.VMEM((gpad, d), jnp.float32)],
    )

    out = pl.pallas_call(
        _seg_kernel,
        grid_spec=grid_spec,
        out_shape=jax.ShapeDtypeStruct((gpad, mlp), jnp.float32),
    )(lo_c, hi_c, n_actual,
      *([nodes] * _K), *([ids_2d] * _K),
      glob_pad, w1, w2, b2)
    return out[:g]
